# initial kernel scaffold (unmeasured)
import jax
import jax.numpy as jnp
from jax import lax
from jax.experimental import pallas as pl
from jax.experimental.pallas import tpu as pltpu


def kernel(
    x,
):
    def body(*refs):
        pass

    out_shape = jax.ShapeDtypeStruct(..., jnp.float32)
    return pl.pallas_call(body, out_shape=out_shape)(...)


# baseline (device time: 17570 ns/iter reference)
import jax
import jax.numpy as jnp
from jax import lax
from jax.experimental import pallas as pl
from jax.experimental.pallas import tpu as pltpu


def kernel(x):
    _, m, n = x.shape
    half = n // 2

    def body(x_ref, out_ref, send_buf, recv_buf, send_sem, recv_sem):
        my_x = lax.axis_index("x")
        my_y = lax.axis_index("y")
        peer = (my_x, 1 - my_y)

        barrier_sem = pltpu.get_barrier_semaphore()
        pl.semaphore_signal(
            barrier_sem, inc=1, device_id=peer,
            device_id_type=pl.DeviceIdType.MESH,
        )
        pl.semaphore_wait(barrier_sem, 1)

        peer_start = (1 - my_y) * half
        my_start = my_y * half
        send_buf[...] = x_ref[0, :, pl.ds(peer_start, half)]
        rdma = pltpu.make_async_remote_copy(
            src_ref=send_buf,
            dst_ref=recv_buf,
            send_sem=send_sem,
            recv_sem=recv_sem,
            device_id=peer,
            device_id_type=pl.DeviceIdType.MESH,
        )
        rdma.start()
        rdma.wait()

        out_ref[...] = x_ref[0, :, pl.ds(my_start, half)] + recv_buf[...]

    return pl.pallas_call(
        body,
        out_shape=jax.ShapeDtypeStruct((m, half), x.dtype),
        in_specs=[pl.BlockSpec(memory_space=pltpu.VMEM)],
        out_specs=pl.BlockSpec(memory_space=pltpu.VMEM),
        scratch_shapes=[
            pltpu.VMEM((m, half), x.dtype),
            pltpu.VMEM((m, half), x.dtype),
            pltpu.SemaphoreType.DMA,
            pltpu.SemaphoreType.DMA,
        ],
        compiler_params=pltpu.CompilerParams(collective_id=0),
    )(x)


# device time: 15802 ns/iter; 1.1119x vs baseline; 1.1119x over previous
import jax
import jax.numpy as jnp
from jax import lax
from jax.experimental import pallas as pl
from jax.experimental.pallas import tpu as pltpu

S = 4


def kernel(x):
    _, m, n = x.shape
    half = n // 2
    mrows = m // 2
    r = mrows // S

    def body(x_ref, out_ref, recv1, sems1_send, sems1_recv, sems2_send, sems2_recv):
        my_x = lax.axis_index("x")
        my_y = lax.axis_index("y")
        ypeer = (my_x, 1 - my_y)
        xpeer = (1 - my_x, my_y)

        my_row0 = my_x * mrows
        other_row0 = (1 - my_x) * mrows
        my_col0 = my_y * half
        peer_col0 = (1 - my_y) * half

        barrier_sem = pltpu.get_barrier_semaphore()
        for nbr in (ypeer, xpeer):
            pl.semaphore_signal(
                barrier_sem, inc=1, device_id=nbr,
                device_id_type=pl.DeviceIdType.MESH,
            )
        pl.semaphore_wait(barrier_sem, 2)

        d1 = []
        for s in range(S):
            d = pltpu.make_async_remote_copy(
                src_ref=x_ref.at[0, pl.ds(my_row0 + s * r, r), pl.ds(peer_col0, half)],
                dst_ref=recv1.at[pl.ds(s * r, r), :],
                send_sem=sems1_send.at[s],
                recv_sem=sems1_recv.at[s],
                device_id=ypeer,
                device_id_type=pl.DeviceIdType.MESH,
            )
            d.start()
            d1.append(d)

        d2 = []
        for s in range(S):
            rows = pl.ds(my_row0 + s * r, r)
            d1[s].wait_recv()
            out_ref[rows, :] = (
                x_ref[0, pl.ds(my_row0 + s * r, r), pl.ds(my_col0, half)]
                + recv1[pl.ds(s * r, r), :]
            )
            d = pltpu.make_async_remote_copy(
                src_ref=out_ref.at[rows, :],
                dst_ref=out_ref.at[rows, :],
                send_sem=sems2_send.at[s],
                recv_sem=sems2_recv.at[s],
                device_id=xpeer,
                device_id_type=pl.DeviceIdType.MESH,
            )
            d.start()
            d2.append(d)

        for s in range(S):
            rows_in = pl.ds(other_row0 + s * r, r)
            recv_desc = pltpu.make_async_remote_copy(
                src_ref=out_ref.at[rows_in, :],
                dst_ref=out_ref.at[rows_in, :],
                send_sem=sems2_send.at[s],
                recv_sem=sems2_recv.at[s],
                device_id=xpeer,
                device_id_type=pl.DeviceIdType.MESH,
            )
            recv_desc.wait_recv()

        for s in range(S):
            d1[s].wait_send()
            d2[s].wait_send()

    return pl.pallas_call(
        body,
        out_shape=jax.ShapeDtypeStruct((m, half), x.dtype),
        in_specs=[pl.BlockSpec(memory_space=pltpu.VMEM)],
        out_specs=pl.BlockSpec(memory_space=pltpu.VMEM),
        scratch_shapes=[
            pltpu.VMEM((mrows, half), x.dtype),
            pltpu.SemaphoreType.DMA((S,)),
            pltpu.SemaphoreType.DMA((S,)),
            pltpu.SemaphoreType.DMA((S,)),
            pltpu.SemaphoreType.DMA((S,)),
        ],
        compiler_params=pltpu.CompilerParams(collective_id=0),
    )(x)


# device time: 15244 ns/iter; 1.1526x vs baseline; 1.0366x over previous
import jax
import jax.numpy as jnp
from jax import lax
from jax.experimental import pallas as pl
from jax.experimental.pallas import tpu as pltpu

S = 8


def kernel(x):
    _, m, n = x.shape
    half = n // 2
    mrows = m // 2
    r = mrows // S

    def body(x_ref, out_ref, recv1, sems1_send, sems1_recv, sems2_send, sems2_recv):
        my_x = lax.axis_index("x")
        my_y = lax.axis_index("y")
        ypeer = (my_x, 1 - my_y)
        xpeer = (1 - my_x, my_y)

        my_row0 = my_x * mrows
        other_row0 = (1 - my_x) * mrows
        my_col0 = my_y * half
        peer_col0 = (1 - my_y) * half

        barrier_sem = pltpu.get_barrier_semaphore()
        for nbr in (ypeer, xpeer):
            pl.semaphore_signal(
                barrier_sem, inc=1, device_id=nbr,
                device_id_type=pl.DeviceIdType.MESH,
            )
        pl.semaphore_wait(barrier_sem, 2)

        d1 = []
        for s in range(S):
            d = pltpu.make_async_remote_copy(
                src_ref=x_ref.at[0, pl.ds(my_row0 + s * r, r), pl.ds(peer_col0, half)],
                dst_ref=recv1.at[pl.ds(s * r, r), :],
                send_sem=sems1_send.at[s],
                recv_sem=sems1_recv.at[s],
                device_id=ypeer,
                device_id_type=pl.DeviceIdType.MESH,
            )
            d.start()
            d1.append(d)

        d2 = []
        for s in range(S):
            rows = pl.ds(my_row0 + s * r, r)
            d1[s].wait_recv()
            out_ref[rows, :] = (
                x_ref[0, pl.ds(my_row0 + s * r, r), pl.ds(my_col0, half)]
                + recv1[pl.ds(s * r, r), :]
            )
            d = pltpu.make_async_remote_copy(
                src_ref=out_ref.at[rows, :],
                dst_ref=out_ref.at[rows, :],
                send_sem=sems2_send.at[s],
                recv_sem=sems2_recv.at[s],
                device_id=xpeer,
                device_id_type=pl.DeviceIdType.MESH,
            )
            d.start()
            d2.append(d)

        for s in range(S):
            rows_in = pl.ds(other_row0 + s * r, r)
            recv_desc = pltpu.make_async_remote_copy(
                src_ref=out_ref.at[rows_in, :],
                dst_ref=out_ref.at[rows_in, :],
                send_sem=sems2_send.at[s],
                recv_sem=sems2_recv.at[s],
                device_id=xpeer,
                device_id_type=pl.DeviceIdType.MESH,
            )
            recv_desc.wait_recv()

        for s in range(S):
            d1[s].wait_send()
            d2[s].wait_send()

    return pl.pallas_call(
        body,
        out_shape=jax.ShapeDtypeStruct((m, half), x.dtype),
        in_specs=[pl.BlockSpec(memory_space=pltpu.VMEM)],
        out_specs=pl.BlockSpec(memory_space=pltpu.VMEM),
        scratch_shapes=[
            pltpu.VMEM((mrows, half), x.dtype),
            pltpu.SemaphoreType.DMA((S,)),
            pltpu.SemaphoreType.DMA((S,)),
            pltpu.SemaphoreType.DMA((S,)),
            pltpu.SemaphoreType.DMA((S,)),
        ],
        compiler_params=pltpu.CompilerParams(collective_id=0),
    )(x)
